# baseline (speedup 1.0000x reference)
import functools

import jax
import jax.numpy as jnp
from jax import lax
from jax.experimental import pallas as pl
from jax.experimental.pallas import tpu as pltpu
from jax.experimental.pallas import tpu_sc as plsc

_NSC = 2
_NSUB = 16
_L = 16
_BIG = 3e38


def _d2_size(shape):
    n = 1
    for v in shape:
        n *= v
    return n


def _make_sc_chamfer(NB, N, M):
    NW = _NSC * _NSUB
    WPB = NW // NB
    QW = N // WPB
    NB_SC = max(1, NB // _NSC)
    RPB = _NSUB // NB_SC
    CS = M // _NSUB
    d2_final = NB >= _NSC
    d2_shape = (NB, M) if d2_final else (_NSC, NB, M)

    mesh = plsc.VectorSubcoreMesh(core_axis_name="c", subcore_axis_name="s")

    @functools.partial(
        pl.kernel, mesh=mesh,
        compiler_params=pltpu.CompilerParams(needs_layout_passes=False),
        out_type=[
            jax.ShapeDtypeStruct((NB * N,), jnp.float32),
            jax.ShapeDtypeStruct((int(jnp.prod(jnp.array(d2_shape))),)
                                 if False else (_d2_size(d2_shape),),
                                 jnp.float32),
        ],
        scratch_types=[
            pltpu.VMEM((QW,), jnp.float32),
            pltpu.VMEM((QW,), jnp.float32),
            pltpu.VMEM((QW,), jnp.float32),
            pltpu.VMEM((M,), jnp.float32),
            pltpu.VMEM((M,), jnp.float32),
            pltpu.VMEM((M,), jnp.float32),
            pltpu.VMEM((QW,), jnp.float32),
            pltpu.VMEM((M,), jnp.float32),
            pltpu.VMEM((16 * 16,), jnp.float32),
            pltpu.VMEM((RPB * CS,), jnp.float32),
            pltpu.VMEM((CS,), jnp.float32),
            pltpu.VMEM_SHARED((_NSUB * M,), jnp.float32),
        ],
    )
    def sc_kernel(x1_hbm, x2_hbm, d1_hbm, d2_hbm,
                  qx, qy, qz, kx, ky, kz, d1v, d2v, accf, cmb, cres, shared):
        c = lax.axis_index("c")
        s = lax.axis_index("s")
        myw = c * _NSUB + s
        batch = myw // WPB
        q0 = (myw % WPB) * QW

        xb1 = batch * (3 * N)
        xb2 = batch * (3 * M)
        pltpu.sync_copy(x1_hbm.at[pl.ds(xb1 + q0, QW)], qx)
        pltpu.sync_copy(x1_hbm.at[pl.ds(xb1 + N + q0, QW)], qy)
        pltpu.sync_copy(x1_hbm.at[pl.ds(xb1 + 2 * N + q0, QW)], qz)
        pltpu.sync_copy(x2_hbm.at[pl.ds(xb2, M)], kx)
        pltpu.sync_copy(x2_hbm.at[pl.ds(xb2 + M, M)], ky)
        pltpu.sync_copy(x2_hbm.at[pl.ds(xb2 + 2 * M, M)], kz)

        def init_body(i, _):
            d2v[pl.ds(i * _L, _L)] = jnp.full((_L,), _BIG, jnp.float32)
            return 0
        lax.fori_loop(0, M // _L, init_body, 0)

        col_iota = lax.iota(jnp.int32, _L)

        def group_body(g, _):
            qbase = g * _L
            qbx = [plsc.load_gather(qx, [jnp.full((_L,), qbase + j, jnp.int32)])
                   for j in range(_L)]
            qby = [plsc.load_gather(qy, [jnp.full((_L,), qbase + j, jnp.int32)])
                   for j in range(_L)]
            qbz = [plsc.load_gather(qz, [jnp.full((_L,), qbase + j, jnp.int32)])
                   for j in range(_L)]

            def kv_body(kv, accs, _KU=2):
                accs = list(accs)
                for u in range(_KU):
                    off = (kv * _KU + u) * _L
                    kxv = kx[pl.ds(off, _L)]
                    kyv = ky[pl.ds(off, _L)]
                    kzv = kz[pl.ds(off, _L)]
                    ds = []
                    for j in range(_L):
                        dx = qbx[j] - kxv
                        dy = qby[j] - kyv
                        dz = qbz[j] - kzv
                        d = dx * dx + dy * dy + dz * dz
                        accs[j] = jnp.minimum(accs[j], d)
                        ds.append(d)
                    while len(ds) > 1:
                        ds = [jnp.minimum(ds[2 * i], ds[2 * i + 1])
                              for i in range(len(ds) // 2)]
                    d2v[pl.ds(off, _L)] = jnp.minimum(d2v[pl.ds(off, _L)],
                                                      ds[0])
                return tuple(accs)

            accs = lax.fori_loop(
                0, M // (_L * 2), kv_body,
                tuple(jnp.full((_L,), _BIG, jnp.float32) for _ in range(_L)))

            for j in range(_L):
                accf[pl.ds(j * _L, _L)] = accs[j]
            r = plsc.load_gather(accf, [col_iota * _L])
            for k in range(1, _L):
                r = jnp.minimum(
                    r, plsc.load_gather(accf, [col_iota * _L + k]))
            d1v[pl.ds(qbase, _L)] = r
            return 0

        lax.fori_loop(0, QW // _L, group_body, 0)

        pltpu.sync_copy(d1v, d1_hbm.at[pl.ds(batch * N + q0, QW)])

        pltpu.sync_copy(d2v, shared.at[pl.ds(s * M, M)])
        plsc.subcore_barrier()

        for lb in range(NB_SC):
            for r in range(RPB):
                pltpu.sync_copy(
                    shared.at[pl.ds((lb * RPB + r) * M + s * CS, CS)],
                    cmb.at[pl.ds(r * CS, CS)])

            def cmb_body(i, _):
                acc = cmb[pl.ds(i * _L, _L)]
                for r in range(1, RPB):
                    acc = jnp.minimum(acc, cmb[pl.ds(r * CS + i * _L, _L)])
                cres[pl.ds(i * _L, _L)] = acc
                return 0
            lax.fori_loop(0, CS // _L, cmb_body, 0)

            if d2_final:
                ob = c * NB_SC + lb
                pltpu.sync_copy(cres, d2_hbm.at[pl.ds(ob * M + s * CS, CS)])
            else:
                pltpu.sync_copy(
                    cres, d2_hbm.at[pl.ds((c * NB + lb) * M + s * CS, CS)])

    return sc_kernel


def sc_chamfer(x1t, x2t):
    NB, _, N = x1t.shape
    M = x2t.shape[2]
    k = _make_sc_chamfer(NB, N, M)
    d1, d2 = k(x1t.reshape(-1), x2t.reshape(-1))
    d1 = d1.reshape(NB, N)
    if NB < _NSC:
        d2 = jnp.min(d2.reshape(_NSC, NB, M), axis=0)
    else:
        d2 = d2.reshape(NB, M)
    return d1, d2


_TN = 2048


def _chamfer_tc_kernel(x1_ref, x2t_ref, d1_ref, d2_ref):
    ni = pl.program_id(1)
    x1 = x1_ref[0]
    x2t = x2t_ref[0]
    dx = x1[:, 0:1] - x2t[0:1, :]
    dy = x1[:, 1:2] - x2t[1:2, :]
    dz = x1[:, 2:3] - x2t[2:3, :]
    d = dx * dx + dy * dy + dz * dz
    d1_ref[0] = jnp.min(d, axis=1, keepdims=True)
    m2 = jnp.min(d, axis=0, keepdims=True)

    @pl.when(ni == 0)
    def _init():
        d2_ref[0] = m2

    @pl.when(ni != 0)
    def _acc():
        d2_ref[0] = jnp.minimum(d2_ref[0], m2)


def tc_chamfer(xyz1, xyz2):
    B, N, _ = xyz1.shape
    M = xyz2.shape[1]
    x1p = jnp.pad(xyz1, ((0, 0), (0, 0), (0, 5)))
    x2t = jnp.transpose(xyz2, (0, 2, 1))
    nt = N // _TN
    d1, d2 = pl.pallas_call(
        _chamfer_tc_kernel,
        grid=(B, nt),
        in_specs=[
            pl.BlockSpec((1, _TN, 8), lambda b, i: (b, i, 0)),
            pl.BlockSpec((1, 3, M), lambda b, i: (b, 0, 0)),
        ],
        out_specs=[
            pl.BlockSpec((1, _TN, 1), lambda b, i: (b, i, 0)),
            pl.BlockSpec((1, 1, M), lambda b, i: (b, 0, 0)),
        ],
        out_shape=[
            jax.ShapeDtypeStruct((B, N, 1), jnp.float32),
            jax.ShapeDtypeStruct((B, 1, M), jnp.float32),
        ],
    )(x1p, x2t)
    return (d1.reshape(B, N), d2.reshape(B, M))


_SCB = 1


@jax.jit
def kernel(input1, input2):
    xyz1 = input1 if input1.shape[2] == 3 else jnp.transpose(input1, (0, 2, 1))
    xyz2 = input2 if input2.shape[2] == 3 else jnp.transpose(input2, (0, 2, 1))
    B = xyz1.shape[0]
    ntc = B - _SCB
    x1t = jnp.transpose(xyz1[ntc:], (0, 2, 1))
    x2t = jnp.transpose(xyz2[ntc:], (0, 2, 1))
    d1_sc, d2_sc = sc_chamfer(x1t, x2t)
    d1_tc, d2_tc = tc_chamfer(xyz1[:ntc], xyz2[:ntc])
    d1 = jnp.concatenate([d1_tc, d1_sc], axis=0)
    d2 = jnp.concatenate([d2_tc, d2_sc], axis=0)
    return (d1, d2)


# speedup vs baseline: 1.4105x; 1.4105x over previous
import functools

import jax
import jax.numpy as jnp
from jax import lax
from jax.experimental import pallas as pl
from jax.experimental.pallas import tpu as pltpu
from jax.experimental.pallas import tpu_sc as plsc

_NSC = 2
_NSUB = 16
_L = 16
_BIG = 3e38


def _d2_size(shape):
    n = 1
    for v in shape:
        n *= v
    return n


def _make_sc_chamfer(NB, N, M):
    NW = _NSC * _NSUB
    WPB = NW // NB
    QW = N // WPB
    NB_SC = max(1, NB // _NSC)
    RPB = _NSUB // NB_SC
    CS = M // _NSUB
    d2_final = NB >= _NSC
    d2_shape = (NB, M) if d2_final else (_NSC, NB, M)

    mesh = plsc.VectorSubcoreMesh(core_axis_name="c", subcore_axis_name="s")

    @functools.partial(
        pl.kernel, mesh=mesh,
        compiler_params=pltpu.CompilerParams(needs_layout_passes=False),
        out_type=[
            jax.ShapeDtypeStruct((NB * N,), jnp.float32),
            jax.ShapeDtypeStruct((int(jnp.prod(jnp.array(d2_shape))),)
                                 if False else (_d2_size(d2_shape),),
                                 jnp.float32),
        ],
        scratch_types=[
            pltpu.VMEM((QW,), jnp.float32),
            pltpu.VMEM((QW,), jnp.float32),
            pltpu.VMEM((QW,), jnp.float32),
            pltpu.VMEM((M,), jnp.float32),
            pltpu.VMEM((M,), jnp.float32),
            pltpu.VMEM((M,), jnp.float32),
            pltpu.VMEM((QW,), jnp.float32),
            pltpu.VMEM((M,), jnp.float32),
            pltpu.VMEM((16 * 16,), jnp.float32),
            pltpu.VMEM((RPB * CS,), jnp.float32),
            pltpu.VMEM((CS,), jnp.float32),
            pltpu.VMEM_SHARED((_NSUB * M,), jnp.float32),
        ],
    )
    def sc_kernel(x1_hbm, x2_hbm, d1_hbm, d2_hbm,
                  qx, qy, qz, kx, ky, kz, d1v, d2v, accf, cmb, cres, shared):
        c = lax.axis_index("c")
        s = lax.axis_index("s")
        myw = c * _NSUB + s
        batch = myw // WPB
        q0 = (myw % WPB) * QW

        xb1 = batch * (3 * N)
        xb2 = batch * (3 * M)
        pltpu.sync_copy(x1_hbm.at[pl.ds(xb1 + q0, QW)], qx)
        pltpu.sync_copy(x1_hbm.at[pl.ds(xb1 + N + q0, QW)], qy)
        pltpu.sync_copy(x1_hbm.at[pl.ds(xb1 + 2 * N + q0, QW)], qz)
        pltpu.sync_copy(x2_hbm.at[pl.ds(xb2, M)], kx)
        pltpu.sync_copy(x2_hbm.at[pl.ds(xb2 + M, M)], ky)
        pltpu.sync_copy(x2_hbm.at[pl.ds(xb2 + 2 * M, M)], kz)

        def init_body(i, _):
            d2v[pl.ds(i * _L, _L)] = jnp.full((_L,), _BIG, jnp.float32)
            return 0
        lax.fori_loop(0, M // _L, init_body, 0)

        col_iota = lax.iota(jnp.int32, _L)

        def group_body(g, _):
            qbase = g * _L
            qbx = [plsc.load_gather(qx, [jnp.full((_L,), qbase + j, jnp.int32)])
                   for j in range(_L)]
            qby = [plsc.load_gather(qy, [jnp.full((_L,), qbase + j, jnp.int32)])
                   for j in range(_L)]
            qbz = [plsc.load_gather(qz, [jnp.full((_L,), qbase + j, jnp.int32)])
                   for j in range(_L)]

            def kv_body(kv, accs, _KU=1):
                accs = list(accs)
                for u in range(_KU):
                    off = (kv * _KU + u) * _L
                    kxv = kx[pl.ds(off, _L)]
                    kyv = ky[pl.ds(off, _L)]
                    kzv = kz[pl.ds(off, _L)]
                    ds = []
                    for j in range(_L):
                        dx = qbx[j] - kxv
                        dy = qby[j] - kyv
                        dz = qbz[j] - kzv
                        d = dx * dx + dy * dy + dz * dz
                        accs[j] = jnp.minimum(accs[j], d)
                        ds.append(d)
                    while len(ds) > 1:
                        ds = [jnp.minimum(ds[2 * i], ds[2 * i + 1])
                              for i in range(len(ds) // 2)]
                    d2v[pl.ds(off, _L)] = jnp.minimum(d2v[pl.ds(off, _L)],
                                                      ds[0])
                return tuple(accs)

            accs = lax.fori_loop(
                0, M // _L, kv_body,
                tuple(jnp.full((_L,), _BIG, jnp.float32) for _ in range(_L)))

            for j in range(_L):
                accf[pl.ds(j * _L, _L)] = accs[j]
            r = plsc.load_gather(accf, [col_iota * _L])
            for k in range(1, _L):
                r = jnp.minimum(
                    r, plsc.load_gather(accf, [col_iota * _L + k]))
            d1v[pl.ds(qbase, _L)] = r
            return 0

        lax.fori_loop(0, QW // _L, group_body, 0)

        pltpu.sync_copy(d1v, d1_hbm.at[pl.ds(batch * N + q0, QW)])

        pltpu.sync_copy(d2v, shared.at[pl.ds(s * M, M)])
        plsc.subcore_barrier()

        for lb in range(NB_SC):
            for r in range(RPB):
                pltpu.sync_copy(
                    shared.at[pl.ds((lb * RPB + r) * M + s * CS, CS)],
                    cmb.at[pl.ds(r * CS, CS)])

            def cmb_body(i, _):
                acc = cmb[pl.ds(i * _L, _L)]
                for r in range(1, RPB):
                    acc = jnp.minimum(acc, cmb[pl.ds(r * CS + i * _L, _L)])
                cres[pl.ds(i * _L, _L)] = acc
                return 0
            lax.fori_loop(0, CS // _L, cmb_body, 0)

            if d2_final:
                ob = c * NB_SC + lb
                pltpu.sync_copy(cres, d2_hbm.at[pl.ds(ob * M + s * CS, CS)])
            else:
                pltpu.sync_copy(
                    cres, d2_hbm.at[pl.ds((c * NB + lb) * M + s * CS, CS)])

    return sc_kernel


def sc_chamfer(x1t, x2t):
    NB, _, N = x1t.shape
    M = x2t.shape[2]
    k = _make_sc_chamfer(NB, N, M)
    d1, d2 = k(x1t.reshape(-1), x2t.reshape(-1))
    d1 = d1.reshape(NB, N)
    if NB < _NSC:
        d2 = jnp.min(d2.reshape(_NSC, NB, M), axis=0)
    else:
        d2 = d2.reshape(NB, M)
    return d1, d2


_TN = 2048


def _chamfer_tc_kernel(x1_ref, x2t_ref, d1_ref, d2_ref):
    ni = pl.program_id(1)
    x1 = x1_ref[0]
    x2t = x2t_ref[0]
    dx = x1[:, 0:1] - x2t[0:1, :]
    dy = x1[:, 1:2] - x2t[1:2, :]
    dz = x1[:, 2:3] - x2t[2:3, :]
    d = dx * dx + dy * dy + dz * dz
    d1_ref[0] = jnp.min(d, axis=1, keepdims=True)
    m2 = jnp.min(d, axis=0, keepdims=True)

    @pl.when(ni == 0)
    def _init():
        d2_ref[0] = m2

    @pl.when(ni != 0)
    def _acc():
        d2_ref[0] = jnp.minimum(d2_ref[0], m2)


def tc_chamfer(xyz1, xyz2):
    B, N, _ = xyz1.shape
    M = xyz2.shape[1]
    x1p = jnp.pad(xyz1, ((0, 0), (0, 0), (0, 5)))
    x2t = jnp.transpose(xyz2, (0, 2, 1))
    nt = N // _TN
    d1, d2 = pl.pallas_call(
        _chamfer_tc_kernel,
        grid=(B, nt),
        in_specs=[
            pl.BlockSpec((1, _TN, 8), lambda b, i: (b, i, 0)),
            pl.BlockSpec((1, 3, M), lambda b, i: (b, 0, 0)),
        ],
        out_specs=[
            pl.BlockSpec((1, _TN, 1), lambda b, i: (b, i, 0)),
            pl.BlockSpec((1, 1, M), lambda b, i: (b, 0, 0)),
        ],
        out_shape=[
            jax.ShapeDtypeStruct((B, N, 1), jnp.float32),
            jax.ShapeDtypeStruct((B, 1, M), jnp.float32),
        ],
    )(x1p, x2t)
    return (d1.reshape(B, N), d2.reshape(B, M))


_SCB = 1


@jax.jit
def kernel(input1, input2):
    xyz1 = input1 if input1.shape[2] == 3 else jnp.transpose(input1, (0, 2, 1))
    xyz2 = input2 if input2.shape[2] == 3 else jnp.transpose(input2, (0, 2, 1))
    B = xyz1.shape[0]
    ntc = B - _SCB
    x1t = jnp.transpose(xyz1[ntc:], (0, 2, 1))
    x2t = jnp.transpose(xyz2[ntc:], (0, 2, 1))
    d1_sc, d2_sc = sc_chamfer(x1t, x2t)
    d1_tc, d2_tc = tc_chamfer(xyz1[:ntc], xyz2[:ntc])
    d1 = jnp.concatenate([d1_tc, d1_sc], axis=0)
    d2 = jnp.concatenate([d2_tc, d2_sc], axis=0)
    return (d1, d2)
